# baseline (speedup 1.0000x reference)
import jax
import jax.numpy as jnp
from jax.experimental import pallas as pl
from jax.experimental.pallas import tpu as pltpu

_K = 8
_V = 4
_TB = 256
_RG = 64

_NEG = -3.0e38


def _kth_largest(l):
    t = jnp.max(l, axis=-1, keepdims=True)
    for _ in range(_K - 1):
        t = jnp.max(jnp.where(l < t, l, _NEG), axis=-1, keepdims=True)
    return t


def _fast_threshold(l):
    q = l.shape[-1]
    rows = l.shape[0]
    t_parts = []
    for r in range(0, rows, _RG):
        lr = l[r:r + _RG]
        s1 = s2 = s3 = jnp.full((_RG, 128), _NEG, jnp.float32)
        for k in range(q // 128):
            xk = lr[:, k * 128:(k + 1) * 128]
            t1 = jnp.maximum(s1, xk)
            b1 = jnp.minimum(s1, xk)
            t2 = jnp.maximum(s2, b1)
            b2 = jnp.minimum(s2, b1)
            s3 = jnp.maximum(s3, b2)
            s1, s2 = t1, t2
        cand = jnp.concatenate([s1, s2, s3], axis=-1)
        t_hat = jnp.max(cand, axis=-1, keepdims=True)
        for _ in range(_K - 1):
            t_hat = jnp.max(jnp.where(cand < t_hat, cand, _NEG),
                            axis=-1, keepdims=True)
        t_parts.append(t_hat)
    return jnp.concatenate(t_parts, axis=0)


def _logits(h, w2_ref, b2_ref, v):
    hv = h[:, v * 64:(v + 1) * 64]
    l = jnp.dot(hv, w2_ref[v], preferred_element_type=jnp.float32)
    return l + b2_ref[v][None, :]


def _body(x_ref, w1_ref, b1_ref, w2_ref, b2_ref, cb_ref,
          dw1_ref, db1_ref, dw2_ref, db2_ref, rec_ref, khot_ref):
    h = jnp.dot(x_ref[...], w1_ref[...], preferred_element_type=jnp.float32)
    h = jnp.maximum(h + b1_ref[...][None, :], 0.0)
    ls = [_logits(h, w2_ref, b2_ref, v) for v in range(_V)]
    khot = None
    bad = jnp.float32(0.0)
    for v in range(_V):
        l = ls[v]
        t_hat = _fast_threshold(l)
        sel = (l >= t_hat).astype(jnp.float32)
        n = jnp.sum(sel, axis=-1, keepdims=True)
        bad = bad + jnp.max(jnp.where(n > jnp.float32(_K) + 0.5, 1.0, 0.0))
        khot = sel if khot is None else jnp.maximum(khot, sel)
    khot_ref[...] = khot

    @pl.when(bad > 0.5)
    def _exact_rescue():
        kh = None
        for v in range(_V):
            l = _logits(h, w2_ref, b2_ref, v)
            sel = (l >= _kth_largest(l)).astype(jnp.float32)
            kh = sel if kh is None else jnp.maximum(kh, sel)
        khot_ref[...] = kh

    kh = khot_ref[...]
    q = jnp.dot(kh, cb_ref[...], preferred_element_type=jnp.float32)
    d = jnp.maximum(
        jnp.dot(q, dw1_ref[...], preferred_element_type=jnp.float32)
        + db1_ref[...][None, :], 0.0)
    rec_ref[...] = (
        jnp.dot(d, dw2_ref[...], preferred_element_type=jnp.float32)
        + db2_ref[...][None, :])


def kernel(x, enc_w1, enc_b1, enc_w2, enc_b2, cb_w,
           dec_w1, dec_b1, dec_w2, dec_b2):
    B, input_dim = x.shape
    V, _, n_hdim = enc_w1.shape
    Q = enc_w2.shape[2]
    n_embd = cb_w.shape[1]
    grid = (B // _TB,)

    w1m = enc_w1.transpose(1, 0, 2).reshape(input_dim, V * n_hdim)
    b1m = enc_b1.reshape(V * n_hdim)

    full = lambda shape: pl.BlockSpec(shape, lambda i: (0,) * len(shape))
    rec, khot = pl.pallas_call(
        _body,
        grid=grid,
        in_specs=[
            pl.BlockSpec((_TB, input_dim), lambda i: (i, 0)),
            full((input_dim, V * n_hdim)),
            full((V * n_hdim,)),
            full((V, n_hdim, Q)),
            full((V, Q)),
            full((Q, n_embd)),
            full((n_embd, n_hdim)),
            full((n_hdim,)),
            full((n_hdim, input_dim)),
            full((input_dim,)),
        ],
        out_specs=[
            pl.BlockSpec((_TB, input_dim), lambda i: (i, 0)),
            pl.BlockSpec((_TB, Q), lambda i: (i, 0)),
        ],
        out_shape=[
            jax.ShapeDtypeStruct((B, input_dim), jnp.float32),
            jax.ShapeDtypeStruct((B, Q), jnp.float32),
        ],
    )(x, w1m, b1m, enc_w2, enc_b2, cb_w, dec_w1, dec_b1, dec_w2, dec_b2)
    return (rec, khot, 0.0)


# speedup vs baseline: 1.0752x; 1.0752x over previous
import jax
import jax.numpy as jnp
from jax.experimental import pallas as pl
from jax.experimental.pallas import tpu as pltpu

_K = 8
_V = 4
_TB = 256
_RG = 64

_NEG = -3.0e38


def _kth_largest(l):
    t = jnp.max(l, axis=-1, keepdims=True)
    for _ in range(_K - 1):
        t = jnp.max(jnp.where(l < t, l, _NEG), axis=-1, keepdims=True)
    return t


def _fast_threshold(l):
    q = l.shape[-1]
    rows = l.shape[0]
    t_parts = []
    for r in range(0, rows, _RG):
        lr = l[r:r + _RG]
        s1 = s2 = s3 = jnp.full((_RG, 128), _NEG, jnp.float32)
        for k in range(q // 128):
            xk = lr[:, k * 128:(k + 1) * 128]
            t1 = jnp.maximum(s1, xk)
            b1 = jnp.minimum(s1, xk)
            t2 = jnp.maximum(s2, b1)
            b2 = jnp.minimum(s2, b1)
            s3 = jnp.maximum(s3, b2)
            s1, s2 = t1, t2
        cand = jnp.concatenate([s1, s2, s3], axis=-1)
        t_hat = jnp.max(cand, axis=-1, keepdims=True)
        for _ in range(_K - 1):
            t_hat = jnp.max(jnp.where(cand < t_hat, cand, _NEG),
                            axis=-1, keepdims=True)
        t_parts.append(t_hat)
    return jnp.concatenate(t_parts, axis=0)


def _logits(h, w2_ref, b2_ref, v):
    hv = h[:, v * 64:(v + 1) * 64]
    l = jnp.dot(hv, w2_ref[v], preferred_element_type=jnp.float32)
    return l + b2_ref[v][None, :]


def _body(x_ref, w1_ref, b1_ref, w2_ref, b2_ref, cb_ref,
          dw1_ref, db1_ref, dw2_ref, db2_ref, rec_ref, khot_ref):
    h = jnp.dot(x_ref[...], w1_ref[...], preferred_element_type=jnp.float32)
    h = jnp.maximum(h + b1_ref[...][None, :], 0.0)
    khot_raw = None
    for v in range(_V):
        l = _logits(h, w2_ref, b2_ref, v)
        t_hat = _fast_threshold(l)
        sel = (l >= t_hat).astype(jnp.float32)
        khot_raw = sel if khot_raw is None else khot_raw + sel
    n = jnp.sum(khot_raw, axis=-1, keepdims=True)
    bad = jnp.max(jnp.where(n > jnp.float32(_V * _K) + 0.5, 1.0, 0.0))
    khot_ref[...] = jnp.minimum(khot_raw, 1.0)

    @pl.when(bad > 0.5)
    def _exact_rescue():
        kh = None
        for v in range(_V):
            l = _logits(h, w2_ref, b2_ref, v)
            sel = (l >= _kth_largest(l)).astype(jnp.float32)
            kh = sel if kh is None else jnp.maximum(kh, sel)
        khot_ref[...] = kh

    kh = khot_ref[...]
    q = jnp.dot(kh, cb_ref[...], preferred_element_type=jnp.float32)
    d = jnp.maximum(
        jnp.dot(q, dw1_ref[...], preferred_element_type=jnp.float32)
        + db1_ref[...][None, :], 0.0)
    rec_ref[...] = (
        jnp.dot(d, dw2_ref[...], preferred_element_type=jnp.float32)
        + db2_ref[...][None, :])


def kernel(x, enc_w1, enc_b1, enc_w2, enc_b2, cb_w,
           dec_w1, dec_b1, dec_w2, dec_b2):
    B, input_dim = x.shape
    V, _, n_hdim = enc_w1.shape
    Q = enc_w2.shape[2]
    n_embd = cb_w.shape[1]
    grid = (B // _TB,)

    w1m = enc_w1.transpose(1, 0, 2).reshape(input_dim, V * n_hdim)
    b1m = enc_b1.reshape(V * n_hdim)

    full = lambda shape: pl.BlockSpec(shape, lambda i: (0,) * len(shape))
    rec, khot = pl.pallas_call(
        _body,
        grid=grid,
        in_specs=[
            pl.BlockSpec((_TB, input_dim), lambda i: (i, 0)),
            full((input_dim, V * n_hdim)),
            full((V * n_hdim,)),
            full((V, n_hdim, Q)),
            full((V, Q)),
            full((Q, n_embd)),
            full((n_embd, n_hdim)),
            full((n_hdim,)),
            full((n_hdim, input_dim)),
            full((input_dim,)),
        ],
        out_specs=[
            pl.BlockSpec((_TB, input_dim), lambda i: (i, 0)),
            pl.BlockSpec((_TB, Q), lambda i: (i, 0)),
        ],
        out_shape=[
            jax.ShapeDtypeStruct((B, input_dim), jnp.float32),
            jax.ShapeDtypeStruct((B, Q), jnp.float32),
        ],
    )(x, w1m, b1m, enc_w2, enc_b2, cb_w, dec_w1, dec_b1, dec_w2, dec_b2)
    return (rec, khot, 0.0)
